# Initial kernel scaffold; baseline (speedup 1.0000x reference)
import jax
import jax.numpy as jnp
from jax.experimental import pallas as pl


def kernel(x, Wr, br, Wn, bn, W1, b1, W2, b2):
    raise NotImplementedError("write your pallas kernel here")


# speedup vs baseline: 1.1862x; 1.1862x over previous
import functools

import jax
import jax.numpy as jnp
from jax.experimental import pallas as pl
from jax.experimental.pallas import tpu as pltpu

D = 1024
E = 8
TOPK = 2
N = 4096
H = 4 * D

HB = 512

_HIGHEST = jax.lax.Precision.HIGHEST


def _dot(a, b, precision):
    return jax.lax.dot_general(a, b, (((1,), (0,)), ((), ())),
                               preferred_element_type=jnp.float32,
                               precision=precision)


def _router_kernel(x_ref, wc_ref, bc_ref, eps_ref, gat_ref):
    xb = x_ref[...]
    lg = _dot(xb, wc_ref[...], None) + bc_ref[...]
    logits = lg[:, :E]
    nlog = lg[:, E:]
    noisy = logits + eps_ref[...] * jax.nn.softplus(nlog)
    idx = jax.lax.broadcasted_iota(jnp.int32, noisy.shape, 1)
    m1 = jnp.max(noisy, axis=1, keepdims=True)
    i1 = jnp.min(jnp.where(noisy == m1, idx, E), axis=1, keepdims=True)
    noisy2 = jnp.where(idx == i1, -jnp.inf, noisy)
    m2 = jnp.max(noisy2, axis=1, keepdims=True)
    i2 = jnp.min(jnp.where(noisy2 == m2, idx, E), axis=1, keepdims=True)
    t = jnp.exp(m2 - m1)
    denom = 1.0 + t
    g1 = 1.0 / denom
    g2 = t / denom
    gat_ref[...] = jnp.where(idx == i1, g1, jnp.where(idx == i2, g2, 0.0))


BN = 1024


def _expert_kernel(gat_ref, x_ref, w1_ref, b1_ref, w2_ref, b2_ref, out_ref):
    e = pl.program_id(1)
    j = pl.program_id(2)

    @pl.when((e == 0) & (j == 0))
    def _init():
        out_ref[...] = jnp.zeros_like(out_ref)

    idx = jax.lax.broadcasted_iota(jnp.int32, (BN, E), 1)
    g = jnp.sum(jnp.where(idx == e, gat_ref[...], 0.0), axis=1, keepdims=True)

    xb = x_ref[...]
    h = jnp.maximum(_dot(xb, w1_ref[0], None) + b1_ref[0], 0.0)
    part = _dot(h, w2_ref[0], None)
    out_ref[...] += part * g

    @pl.when(j == 0)
    def _bias():
        out_ref[...] += b2_ref[0] * g


def kernel(x, Wr, br, Wn, bn, W1, b1, W2, b2):
    wc = jnp.concatenate([Wr, Wn], axis=1)
    bc = jnp.concatenate([br, bn])[None, :]
    eps = jax.random.normal(jax.random.key(42), (N, E), dtype=jnp.float32)

    gating = pl.pallas_call(
        _router_kernel,
        out_shape=jax.ShapeDtypeStruct((N, E), jnp.float32),
        in_specs=[
            pl.BlockSpec((N, D), lambda: (0, 0)),
            pl.BlockSpec((D, 2 * E), lambda: (0, 0)),
            pl.BlockSpec((1, 2 * E), lambda: (0, 0)),
            pl.BlockSpec((N, E), lambda: (0, 0)),
        ],
        out_specs=pl.BlockSpec((N, E), lambda: (0, 0)),
    )(x, wc, bc, eps)

    b1r = b1.reshape(E, 1, H)
    b2r = b2.reshape(E, 1, D)
    out = pl.pallas_call(
        _expert_kernel,
        grid=(N // BN, E, H // HB),
        out_shape=jax.ShapeDtypeStruct((N, D), jnp.float32),
        in_specs=[
            pl.BlockSpec((BN, E), lambda i, e, j: (i, 0)),
            pl.BlockSpec((BN, D), lambda i, e, j: (i, 0)),
            pl.BlockSpec((1, D, HB), lambda i, e, j: (e, 0, j)),
            pl.BlockSpec((1, 1, HB), lambda i, e, j: (e, 0, j)),
            pl.BlockSpec((1, HB, D), lambda i, e, j: (e, j, 0)),
            pl.BlockSpec((1, 1, D), lambda i, e, j: (e, 0, 0)),
        ],
        out_specs=pl.BlockSpec((BN, D), lambda i, e, j: (i, 0)),
    )(gating, x, W1, b1r, W2, b2r)
    return out
